# initial kernel scaffold (unmeasured)
import jax
import jax.numpy as jnp
from jax import lax
from jax.experimental import pallas as pl
from jax.experimental.pallas import tpu as pltpu

N_DEV = 8
SQ = 1024
SKV = 1024
H_LOC = 8
DH = 128
D_LOC = H_LOC * DH
BLK = 64
SCALE = 0.08838834764831843


def kernel(x, Wq, K_ext, V_ext, Wo):
    my = lax.axis_index("i")
    Wq_loc = lax.dynamic_slice(Wq, (0, my * D_LOC), (Wq.shape[0], D_LOC))
    Wo_loc = lax.dynamic_slice(Wo, (my * D_LOC, 0), (D_LOC, Wo.shape[1]))

    def body(x_ref, wq_ref, k_ref, v_ref, wo_ref, out_ref,
             comm_ref, send_sems, recv_sems):
        my_pos = lax.axis_index("i")
        left = lax.rem(my_pos - 1 + N_DEV, N_DEV)
        right = lax.rem(my_pos + 1, N_DEV)

        barrier_sem = pltpu.get_barrier_semaphore()
        for nbr in (left, right):
            pl.semaphore_signal(barrier_sem, inc=1, device_id=(nbr,),
                                device_id_type=pl.DeviceIdType.MESH)
        pl.semaphore_wait(barrier_sem, 2)

        xm = x_ref[0]
        Q = jnp.dot(xm, wq_ref[...], preferred_element_type=jnp.float32)

        rb = lax.broadcasted_iota(jnp.int32, (SQ, SKV), 0) // BLK
        cb = lax.broadcasted_iota(jnp.int32, (SQ, SKV), 1) // BLK
        mask = cb <= rb

        ctx_parts = []
        for h in range(H_LOC):
            q = Q[:, h * DH:(h + 1) * DH]
            k = k_ref[0, :, h, :]
            v = v_ref[0, :, h, :]
            s = lax.dot_general(q, k, (((1,), (1,)), ((), ())),
                                preferred_element_type=jnp.float32) * SCALE
            s = jnp.where(mask, s, -1e9)
            m = jnp.max(s, axis=-1, keepdims=True)
            w = jnp.exp(s - m)
            w = w / jnp.sum(w, axis=-1, keepdims=True)
            ctx_parts.append(jnp.dot(w, v, preferred_element_type=jnp.float32))
        ctx = jnp.concatenate(ctx_parts, axis=1)
        partial = jnp.dot(ctx, wo_ref[...], preferred_element_type=jnp.float32)

        out_ref[0] = partial
        comm_ref[0] = partial

        for hop in range(N_DEV - 1):
            rdma = pltpu.make_async_remote_copy(
                src_ref=comm_ref.at[hop],
                dst_ref=comm_ref.at[hop + 1],
                send_sem=send_sems.at[hop],
                recv_sem=recv_sems.at[hop],
                device_id=(right,),
                device_id_type=pl.DeviceIdType.MESH,
            )
            rdma.start()
            rdma.wait()
            out_ref[0] += comm_ref[hop + 1]

    return pl.pallas_call(
        body,
        out_shape=jax.ShapeDtypeStruct((1, SQ, Wo.shape[1]), jnp.float32),
        in_specs=[pl.BlockSpec(memory_space=pltpu.VMEM)] * 5,
        out_specs=pl.BlockSpec(memory_space=pltpu.VMEM),
        scratch_shapes=[
            pltpu.VMEM((N_DEV, SQ, 1024), jnp.float32),
            pltpu.SemaphoreType.DMA((N_DEV - 1,)),
            pltpu.SemaphoreType.DMA((N_DEV - 1,)),
        ],
        compiler_params=pltpu.CompilerParams(collective_id=0),
    )(x, Wq_loc, K_ext, V_ext, Wo_loc)


# baseline (device time: 367986 ns/iter reference)
import jax
import jax.numpy as jnp
from jax import lax
from jax.experimental import pallas as pl
from jax.experimental.pallas import tpu as pltpu

N_DEV = 8
SQ = 1024
SKV = 1024
H_LOC = 8
DH = 128
D_LOC = H_LOC * DH
BLK = 64
SCALE = 0.08838834764831843


def kernel(x, Wq, K_ext, V_ext, Wo):
    my = lax.axis_index("i")
    Wq_loc = lax.dynamic_slice(Wq, (0, my * D_LOC), (Wq.shape[0], D_LOC))
    Wo_loc = lax.dynamic_slice(Wo, (my * D_LOC, 0), (D_LOC, Wo.shape[1]))

    def body(x_ref, wq_ref, k_ref, v_ref, wo_ref, out_ref,
             comm_ref, send_sems, recv_sems):
        my_pos = lax.axis_index("i")
        left = lax.rem(my_pos - 1 + N_DEV, N_DEV)
        right = lax.rem(my_pos + 1, N_DEV)

        barrier_sem = pltpu.get_barrier_semaphore()
        for nbr in (left, right):
            pl.semaphore_signal(barrier_sem, inc=1, device_id=(nbr,),
                                device_id_type=pl.DeviceIdType.MESH)
        pl.semaphore_wait(barrier_sem, 2)

        xm = x_ref[0]
        Q = jnp.dot(xm, wq_ref[...], preferred_element_type=jnp.float32)

        rb = lax.broadcasted_iota(jnp.int32, (SQ, SKV), 0) // BLK
        cb = lax.broadcasted_iota(jnp.int32, (SQ, SKV), 1) // BLK
        mask = cb <= rb

        ctx_parts = []
        for h in range(H_LOC):
            q = Q[:, h * DH:(h + 1) * DH]
            k = k_ref[0, :, h, :]
            v = v_ref[0, :, h, :]
            s = lax.dot_general(q, k, (((1,), (1,)), ((), ())),
                                preferred_element_type=jnp.float32) * SCALE
            s = jnp.where(mask, s, -1e9)
            m = jnp.max(s, axis=-1, keepdims=True)
            w = jnp.exp(s - m)
            w = w / jnp.sum(w, axis=-1, keepdims=True)
            ctx_parts.append(jnp.dot(w, v, preferred_element_type=jnp.float32))
        ctx = jnp.concatenate(ctx_parts, axis=1)
        partial = jnp.dot(ctx, wo_ref[...], preferred_element_type=jnp.float32)

        out_ref[0] = partial
        comm_ref[0] = partial

        for hop in range(N_DEV - 1):
            send_slot = hop % 2
            recv_slot = (hop + 1) % 2
            rdma = pltpu.make_async_remote_copy(
                src_ref=comm_ref.at[send_slot],
                dst_ref=comm_ref.at[recv_slot],
                send_sem=send_sems.at[hop],
                recv_sem=recv_sems.at[hop],
                device_id=(right,),
                device_id_type=pl.DeviceIdType.MESH,
            )
            rdma.start()
            rdma.wait()
            out_ref[0] += comm_ref[recv_slot]

    return pl.pallas_call(
        body,
        out_shape=jax.ShapeDtypeStruct((1, SQ, Wo.shape[1]), jnp.float32),
        in_specs=[pl.BlockSpec(memory_space=pltpu.VMEM)] * 5,
        out_specs=pl.BlockSpec(memory_space=pltpu.VMEM),
        scratch_shapes=[
            pltpu.VMEM((2, SQ, 1024), jnp.float32),
            pltpu.SemaphoreType.DMA((N_DEV - 1,)),
            pltpu.SemaphoreType.DMA((N_DEV - 1,)),
        ],
        compiler_params=pltpu.CompilerParams(collective_id=0),
    )(x, Wq_loc, K_ext, V_ext, Wo_loc)


# device time: 108055 ns/iter; 3.4055x vs baseline; 3.4055x over previous
import jax
import jax.numpy as jnp
from jax import lax
from jax.experimental import pallas as pl
from jax.experimental.pallas import tpu as pltpu

N_DEV = 8
SQ = 1024
SKV = 1024
H_LOC = 8
DH = 128
D_LOC = H_LOC * DH
BLK = 64
SCALE = 0.08838834764831843


def kernel(x, Wq, K_ext, V_ext, Wo):
    my = lax.axis_index("i")
    Wq_loc = lax.dynamic_slice(Wq, (0, my * D_LOC), (Wq.shape[0], D_LOC))
    Wo_loc = lax.dynamic_slice(Wo, (my * D_LOC, 0), (D_LOC, Wo.shape[1]))

    def body(x_ref, wq_ref, k_ref, v_ref, wo_ref, out_ref,
             comm_ref, rs_send, rs_recv, ag_send, ag_recv):
        my_pos = lax.axis_index("i")
        left = lax.rem(my_pos - 1 + N_DEV, N_DEV)
        right = lax.rem(my_pos + 1, N_DEV)

        barrier_sem = pltpu.get_barrier_semaphore()
        for nbr in (left, right):
            pl.semaphore_signal(barrier_sem, inc=1, device_id=(nbr,),
                                device_id_type=pl.DeviceIdType.MESH)
        pl.semaphore_wait(barrier_sem, 2)

        xm = x_ref[0]
        Q = jnp.dot(xm, wq_ref[...], preferred_element_type=jnp.float32)

        rb = lax.broadcasted_iota(jnp.int32, (SQ, SKV), 0) // BLK
        cb = lax.broadcasted_iota(jnp.int32, (SQ, SKV), 1) // BLK
        mask = cb <= rb

        ctx_parts = []
        for h in range(H_LOC):
            q = Q[:, h * DH:(h + 1) * DH]
            k = k_ref[0, :, h, :]
            v = v_ref[0, :, h, :]
            s = lax.dot_general(q, k, (((1,), (1,)), ((), ())),
                                preferred_element_type=jnp.float32) * SCALE
            s = jnp.where(mask, s, -1e9)
            m = jnp.max(s, axis=-1, keepdims=True)
            w = jnp.exp(s - m)
            w = w / jnp.sum(w, axis=-1, keepdims=True)
            ctx_parts.append(jnp.dot(w, v, preferred_element_type=jnp.float32))
        ctx = jnp.concatenate(ctx_parts, axis=1)
        partial = jnp.dot(ctx, wo_ref[...], preferred_element_type=jnp.float32)

        out_ref[0] = partial

        HALF = 512
        R = SQ // N_DEV

        def rows(c):
            return pl.ds(c * R, R)

        def cols(dirn):
            return pl.ds(dirn * HALF, HALF)

        for h in range(N_DEV - 1):
            rdmas = []
            for dirn in range(2):
                nbr = right if dirn == 0 else left
                if h == 0:
                    src = out_ref.at[0, rows(my_pos), cols(dirn)]
                else:
                    src = comm_ref.at[dirn, h % 2]
                rdma = pltpu.make_async_remote_copy(
                    src_ref=src,
                    dst_ref=comm_ref.at[dirn, (h + 1) % 2],
                    send_sem=rs_send.at[dirn, h],
                    recv_sem=rs_recv.at[dirn, h],
                    device_id=(nbr,),
                    device_id_type=pl.DeviceIdType.MESH,
                )
                rdma.start()
                rdmas.append(rdma)
            for dirn in range(2):
                rdmas[dirn].wait()
                if dirn == 0:
                    c = lax.rem(my_pos - h - 1 + N_DEV, N_DEV)
                else:
                    c = lax.rem(my_pos + h + 1, N_DEV)
                acc = comm_ref[dirn, (h + 1) % 2] + out_ref[0, rows(c), cols(dirn)]
                if h < N_DEV - 2:
                    comm_ref[dirn, (h + 1) % 2] = acc
                else:
                    out_ref[0, rows(c), cols(dirn)] = acc

        for k in range(N_DEV - 1):
            rdmas = []
            for dirn in range(2):
                nbr = right if dirn == 0 else left
                if dirn == 0:
                    c_send = lax.rem(my_pos + 1 - k + N_DEV, N_DEV)
                else:
                    c_send = lax.rem(my_pos - 1 + k + N_DEV, N_DEV)
                rdma = pltpu.make_async_remote_copy(
                    src_ref=out_ref.at[0, rows(c_send), cols(dirn)],
                    dst_ref=out_ref.at[0, rows(c_send), cols(dirn)],
                    send_sem=ag_send.at[dirn, k],
                    recv_sem=ag_recv.at[dirn, k],
                    device_id=(nbr,),
                    device_id_type=pl.DeviceIdType.MESH,
                )
                rdma.start()
                rdmas.append(rdma)
            for dirn in range(2):
                rdmas[dirn].wait()

    return pl.pallas_call(
        body,
        out_shape=jax.ShapeDtypeStruct((1, SQ, Wo.shape[1]), jnp.float32),
        in_specs=[pl.BlockSpec(memory_space=pltpu.VMEM)] * 5,
        out_specs=pl.BlockSpec(memory_space=pltpu.VMEM),
        scratch_shapes=[
            pltpu.VMEM((2, 2, SQ // N_DEV, 512), jnp.float32),
            pltpu.SemaphoreType.DMA((2, N_DEV - 1)),
            pltpu.SemaphoreType.DMA((2, N_DEV - 1)),
            pltpu.SemaphoreType.DMA((2, N_DEV - 1)),
            pltpu.SemaphoreType.DMA((2, N_DEV - 1)),
        ],
        compiler_params=pltpu.CompilerParams(collective_id=0),
    )(x, Wq_loc, K_ext, V_ext, Wo_loc)
